# baseline (speedup 1.0000x reference)
import math

import jax
import jax.numpy as jnp
import numpy as np
from jax.experimental import pallas as pl
from jax.experimental.pallas import tpu as pltpu


def _norm_alpha(sample_rate=8000, hop_size=80, norm_tau=1.0):
    a_ = math.exp(-(hop_size / sample_rate) / norm_tau)
    precision = 3
    a = 1.0
    while a >= 1.0:
        a = round(a_, precision)
        precision += 1
    return a


_ALPHA = _norm_alpha()
_LC = 128
_T_BLK = 16384
_ROW_BLK = 64


def _chunk_operator(alpha, n):
    j = np.arange(n)[:, None]
    i = np.arange(n)[None, :]
    a = np.where(j <= i, (1.0 - alpha) * np.power(alpha, i - j), 0.0)
    b = (1.0 - alpha) * np.power(alpha, n - 1 - j) * np.power(alpha, i + 1)
    return np.concatenate([a, b], axis=1).astype(np.float32)


def _ema_body(x_ref, ab_ref, s0_ref, o_ref, fs_ref):
    tstep = pl.program_id(1)

    @pl.when(tstep == 0)
    def _init():
        fs_ref[...] = s0_ref[...]

    ab = ab_ref[...].astype(jnp.bfloat16)

    lane = jax.lax.broadcasted_iota(jnp.int32, (1, _LC), 1).astype(jnp.float32)
    prow = jnp.exp(np.float32(math.log(_ALPHA)) * (lane + 1.0))
    decay = np.float32(_ALPHA ** _LC)

    h = fs_ref[...]
    hterm = h * prow

    n_chunks = _T_BLK // _LC
    s = None
    for c in range(n_chunks):
        xc = x_ref[:, pl.ds(c * _LC, _LC)]
        r = jnp.dot(xc.astype(jnp.bfloat16), ab,
                    preferred_element_type=jnp.float32)
        s = r[:, :_LC] + hterm
        o_ref[:, pl.ds(c * _LC, _LC)] = (xc - s) * np.float32(1.0 / 40.0)
        hterm = r[:, _LC:] + decay * hterm

    fs_ref[...] = s[:, _LC - 1:_LC]


def kernel(feat_erb, state):
    b, c, t, f = feat_erb.shape
    rows = b * c * f
    x = jnp.transpose(feat_erb, (0, 1, 3, 2)).reshape(rows, t)
    s0 = jnp.tile(state.astype(feat_erb.dtype).reshape(c * f), (b,)).reshape(rows, 1)
    ab = jnp.asarray(_chunk_operator(_ALPHA, _LC))

    grid = (rows // _ROW_BLK, t // _T_BLK)
    out, fstate = pl.pallas_call(
        _ema_body,
        grid=grid,
        in_specs=[
            pl.BlockSpec((_ROW_BLK, _T_BLK), lambda i, j: (i, j)),
            pl.BlockSpec((_LC, 2 * _LC), lambda i, j: (0, 0)),
            pl.BlockSpec((_ROW_BLK, 1), lambda i, j: (i, 0)),
        ],
        out_specs=[
            pl.BlockSpec((_ROW_BLK, _T_BLK), lambda i, j: (i, j)),
            pl.BlockSpec((_ROW_BLK, 1), lambda i, j: (i, 0)),
        ],
        out_shape=[
            jax.ShapeDtypeStruct((rows, t), feat_erb.dtype),
            jax.ShapeDtypeStruct((rows, 1), feat_erb.dtype),
        ],
        compiler_params=pltpu.CompilerParams(
            dimension_semantics=("parallel", "arbitrary"),
            vmem_limit_bytes=56 * 1024 * 1024,
        ),
        name="erb_ema",
    )(x, ab, s0)

    feat_out = jnp.transpose(out.reshape(b, c, f, t), (0, 1, 3, 2))
    return feat_out, fstate.reshape(b, c, f)


# speedup vs baseline: 1.0223x; 1.0223x over previous
import math

import jax
import jax.numpy as jnp
import numpy as np
from jax.experimental import pallas as pl
from jax.experimental.pallas import tpu as pltpu


def _norm_alpha(sample_rate=8000, hop_size=80, norm_tau=1.0):
    a_ = math.exp(-(hop_size / sample_rate) / norm_tau)
    precision = 3
    a = 1.0
    while a >= 1.0:
        a = round(a_, precision)
        precision += 1
    return a


_ALPHA = _norm_alpha()
_LC = 128
_T_BLK = 16384
_ROW_BLK = 128


def _chunk_operator(alpha, n):
    j = np.arange(n)[:, None]
    i = np.arange(n)[None, :]
    a = np.where(j <= i, (1.0 - alpha) * np.power(alpha, i - j), 0.0)
    b = (1.0 - alpha) * np.power(alpha, n - 1 - j) * np.power(alpha, i + 1)
    return np.concatenate([a, b], axis=1).astype(np.float32)


def _ema_body(x_ref, ab_ref, s0_ref, o_ref, fs_ref):
    tstep = pl.program_id(1)

    @pl.when(tstep == 0)
    def _init():
        fs_ref[...] = s0_ref[...]

    ab = ab_ref[...].astype(jnp.bfloat16)

    lane = jax.lax.broadcasted_iota(jnp.int32, (1, _LC), 1).astype(jnp.float32)
    prow = jnp.exp(np.float32(math.log(_ALPHA)) * (lane + 1.0))
    decay = np.float32(_ALPHA ** _LC)

    h = fs_ref[...]
    hterm = h * prow

    n_chunks = _T_BLK // _LC
    s = None
    for c in range(n_chunks):
        xc = x_ref[:, pl.ds(c * _LC, _LC)]
        r = jnp.dot(xc.astype(jnp.bfloat16), ab,
                    preferred_element_type=jnp.float32)
        s = r[:, :_LC] + hterm
        o_ref[:, pl.ds(c * _LC, _LC)] = (xc - s) * np.float32(1.0 / 40.0)
        hterm = r[:, _LC:] + decay * hterm

    fs_ref[...] = s[:, _LC - 1:_LC]


def kernel(feat_erb, state):
    b, c, t, f = feat_erb.shape
    rows = b * c * f
    x = jnp.transpose(feat_erb, (0, 1, 3, 2)).reshape(rows, t)
    s0 = jnp.tile(state.astype(feat_erb.dtype).reshape(c * f), (b,)).reshape(rows, 1)
    ab = jnp.asarray(_chunk_operator(_ALPHA, _LC))

    grid = (rows // _ROW_BLK, t // _T_BLK)
    out, fstate = pl.pallas_call(
        _ema_body,
        grid=grid,
        in_specs=[
            pl.BlockSpec((_ROW_BLK, _T_BLK), lambda i, j: (i, j)),
            pl.BlockSpec((_LC, 2 * _LC), lambda i, j: (0, 0)),
            pl.BlockSpec((_ROW_BLK, 1), lambda i, j: (i, 0)),
        ],
        out_specs=[
            pl.BlockSpec((_ROW_BLK, _T_BLK), lambda i, j: (i, j)),
            pl.BlockSpec((_ROW_BLK, 1), lambda i, j: (i, 0)),
        ],
        out_shape=[
            jax.ShapeDtypeStruct((rows, t), feat_erb.dtype),
            jax.ShapeDtypeStruct((rows, 1), feat_erb.dtype),
        ],
        compiler_params=pltpu.CompilerParams(
            dimension_semantics=("parallel", "arbitrary"),
            vmem_limit_bytes=56 * 1024 * 1024,
        ),
        name="erb_ema",
    )(x, ab, s0)

    feat_out = jnp.transpose(out.reshape(b, c, f, t), (0, 1, 3, 2))
    return feat_out, fstate.reshape(b, c, f)
